# Initial kernel scaffold; baseline (speedup 1.0000x reference)
import functools

import jax
import jax.numpy as jnp
from jax import lax
from jax.experimental import pallas as pl
from jax.experimental.pallas import tpu as pltpu
from jax.experimental.pallas import tpu_sc as plsc

D = 1024
E = 64
K = 8
G = 8
KG = 4
F = 512
FS = 1024
T = 2048
P = T * K
BLK = 128
PAD = P + E * BLK
NBLK = PAD // BLK
TB = 256

NC, NS = 2, 16
NW = NC * NS


def _gate_body(x_ref, gwt_ref, w_ref, idx_ref):
    xb = x_ref[...]
    logits = jnp.dot(xb, gwt_ref[...], preferred_element_type=jnp.float32)
    m = jnp.max(logits, axis=-1, keepdims=True)
    ex = jnp.exp(logits - m)
    scores = ex / jnp.sum(ex, axis=-1, keepdims=True)

    lane64 = lax.broadcasted_iota(jnp.int32, (TB, E), 1)
    gid = lane64 // G
    neg = jnp.float32(-jnp.inf)

    gs_cols = []
    for g in range(G):
        gs_cols.append(jnp.max(jnp.where(gid == g, scores, neg), axis=-1,
                               keepdims=True))
    lane8 = lax.broadcasted_iota(jnp.int32, (TB, G), 1)
    gs = jnp.zeros((TB, G), jnp.float32)
    for g in range(G):
        gs = jnp.where(lane8 == g, gs_cols[g], gs)

    gmask = jnp.zeros((TB, G), jnp.bool_)
    gm = gs
    big8 = jnp.int32(G + 1)
    for _ in range(KG):
        mx = jnp.max(gm, axis=-1, keepdims=True)
        sel = jnp.min(jnp.where(gm == mx, lane8, big8), axis=-1, keepdims=True)
        oh = lane8 == sel
        gmask = jnp.logical_or(gmask, oh)
        gm = jnp.where(oh, neg, gm)

    emask = jnp.zeros((TB, E), jnp.bool_)
    for g in range(G):
        emask = jnp.where(gid == g, gmask[:, g:g + 1], emask)

    masked = jnp.where(emask, scores, 0.0)
    big64 = jnp.int32(E + 1)
    wout = jnp.zeros((TB, K), jnp.float32)
    iout = jnp.zeros((TB, K), jnp.int32)
    lane_k = lax.broadcasted_iota(jnp.int32, (TB, K), 1)
    mm = masked
    for k in range(K):
        mx = jnp.max(mm, axis=-1, keepdims=True)
        sel = jnp.min(jnp.where(mm == mx, lane64, big64), axis=-1,
                      keepdims=True)
        oh = lane64 == sel
        wk = jnp.sum(jnp.where(oh, scores, 0.0), axis=-1, keepdims=True)
        mm = jnp.where(oh, neg, mm)
        wout = jnp.where(lane_k == k, wk, wout)
        iout = jnp.where(lane_k == k, sel, iout)
    w_ref[...] = wout
    idx_ref[...] = iout


def _gate(x, gate_w):
    gwt = gate_w.T
    return pl.pallas_call(
        _gate_body,
        grid=(T // TB,),
        in_specs=[
            pl.BlockSpec((TB, D), lambda i: (i, 0)),
            pl.BlockSpec((D, E), lambda i: (0, 0)),
        ],
        out_specs=[
            pl.BlockSpec((TB, K), lambda i: (i, 0)),
            pl.BlockSpec((TB, K), lambda i: (i, 0)),
        ],
        out_shape=[
            jax.ShapeDtypeStruct((T, K), jnp.float32),
            jax.ShapeDtypeStruct((T, K), jnp.int32),
        ],
    )(x, gwt)


CHUNK = 512
NCHUNK = P // CHUNK


def _slots_body(idx_ref, pos_ref, bexp_ref, nb_ref, part_ref):
    r = lax.broadcasted_iota(jnp.int32, (CHUNK, CHUNK), 0)
    c = lax.broadcasted_iota(jnp.int32, (CHUNK, CHUNK), 1)
    L = (r > c).astype(jnp.float32)
    lane64c = lax.broadcasted_iota(jnp.int32, (CHUNK, E), 1)

    def pass1(ci, run):
        sl = pl.ds(ci * CHUNK, CHUNK)
        oh = (idx_ref[sl, :] == lane64c).astype(jnp.float32)
        rank = jnp.dot(L, oh, preferred_element_type=jnp.float32)
        pick_rank = jnp.sum(rank * oh, axis=-1, keepdims=True)
        pick_prior = jnp.sum(run * oh, axis=-1, keepdims=True)
        part_ref[sl, :] = pick_rank + pick_prior
        return run + jnp.sum(oh, axis=0, keepdims=True)

    counts = lax.fori_loop(0, NCHUNK, pass1,
                           jnp.zeros((1, E), jnp.float32))

    counts_i = counts.astype(jnp.int32)
    padded_i = (counts_i + (BLK - 1)) // BLK * BLK
    padded = padded_i.astype(jnp.float32)
    ru = lax.broadcasted_iota(jnp.int32, (E, E), 0)
    cu = lax.broadcasted_iota(jnp.int32, (E, E), 1)
    U = (ru < cu).astype(jnp.float32)
    starts = jnp.dot(padded, U, preferred_element_type=jnp.float32)

    def pass2(ci, carry):
        sl = pl.ds(ci * CHUNK, CHUNK)
        oh = (idx_ref[sl, :] == lane64c).astype(jnp.float32)
        pick_start = jnp.sum(starts * oh, axis=-1, keepdims=True)
        pos_ref[sl, :] = (part_ref[sl, :] + pick_start).astype(jnp.int32)
        return carry

    lax.fori_loop(0, NCHUNK, pass2, 0)

    ends = starts + padded
    srow = lax.broadcasted_iota(jnp.float32, (NBLK, E), 0) * BLK
    cnt = jnp.sum((ends <= srow).astype(jnp.float32), axis=-1,
                  keepdims=True)
    bexp_ref[...] = jnp.minimum(cnt.astype(jnp.int32), E - 1)
    nb_ref[...] = (jnp.sum(padded, axis=-1, keepdims=True)
                   / BLK).astype(jnp.int32)


def _slots(idx_col):
    return pl.pallas_call(
        _slots_body,
        out_shape=[
            jax.ShapeDtypeStruct((P, 1), jnp.int32),
            jax.ShapeDtypeStruct((NBLK, 1), jnp.int32),
            jax.ShapeDtypeStruct((1, 1), jnp.int32),
        ],
        scratch_shapes=[pltpu.VMEM((P, 1), jnp.float32)],
    )(idx_col)


CH3 = 32
PPW = P // NW
NCH3 = PPW // CH3


def _dispatch_body(x_hbm, tok_hbm, slot_hbm, xs_hbm,
                   tok_v, slot_v, rows_v, sem_g, sem_s):
    wid = lax.axis_index("s") * NC + lax.axis_index("c")
    base = wid * PPW

    def chunk(i, carry):
        off = pl.multiple_of(base + i * CH3, CH3)
        pltpu.sync_copy(tok_hbm.at[pl.ds(off, CH3)], tok_v)
        pltpu.sync_copy(slot_hbm.at[pl.ds(off, CH3)], slot_v)
        pltpu.async_copy(x_hbm.at[tok_v], rows_v, sem_g).wait()
        pltpu.async_copy(rows_v, xs_hbm.at[slot_v], sem_s).wait()
        return carry

    lax.fori_loop(0, NCH3, chunk, 0)


_dispatch = functools.partial(
    pl.kernel,
    out_type=jax.ShapeDtypeStruct((PAD, D), jnp.float32),
    mesh=plsc.VectorSubcoreMesh(core_axis_name="c", subcore_axis_name="s",
                                num_cores=NC, num_subcores=NS),
    scratch_types=[
        pltpu.VMEM((CH3,), jnp.int32),
        pltpu.VMEM((CH3,), jnp.int32),
        pltpu.VMEM((CH3, D), jnp.float32),
        pltpu.SemaphoreType.DMA,
        pltpu.SemaphoreType.DMA,
    ],
)(_dispatch_body)


def _expert_body(bexp_ref, nb_ref, xs_ref, w1_ref, w3_ref, w2_ref, ys_ref):
    b = pl.program_id(0)

    @pl.when(b < nb_ref[0])
    def _():
        xb = xs_ref[...]
        w1 = w1_ref[0]
        w3 = w3_ref[0]
        w2 = w2_ref[0]
        dn = (((1,), (1,)), ((), ()))
        a = lax.dot_general(xb, w1, dn, preferred_element_type=jnp.float32)
        bq = lax.dot_general(xb, w3, dn, preferred_element_type=jnp.float32)
        h = a * jax.nn.sigmoid(a) * bq
        ys_ref[...] = lax.dot_general(h, w2, dn,
                                      preferred_element_type=jnp.float32)


def _experts(xs, W1, W3, W2, bexp, nb):
    def clamp(b, nb_):
        return jnp.minimum(b, nb_[0] - 1)

    grid_spec = pltpu.PrefetchScalarGridSpec(
        num_scalar_prefetch=2,
        grid=(NBLK,),
        in_specs=[
            pl.BlockSpec((BLK, D), lambda b, be, nb_: (clamp(b, nb_), 0)),
            pl.BlockSpec((1, F, D),
                         lambda b, be, nb_: (be[clamp(b, nb_)], 0, 0)),
            pl.BlockSpec((1, F, D),
                         lambda b, be, nb_: (be[clamp(b, nb_)], 0, 0)),
            pl.BlockSpec((1, D, F),
                         lambda b, be, nb_: (be[clamp(b, nb_)], 0, 0)),
        ],
        out_specs=pl.BlockSpec((BLK, D),
                               lambda b, be, nb_: (clamp(b, nb_), 0)),
    )
    return pl.pallas_call(
        _expert_body,
        grid_spec=grid_spec,
        out_shape=jax.ShapeDtypeStruct((PAD, D), jnp.float32),
    )(bexp, nb, xs, W1, W3, W2)


SB = 128


def _shared_body(x_ref, w1_ref, w3_ref, w2_ref, z_ref):
    xb = x_ref[...]
    dn = (((1,), (1,)), ((), ()))
    a = lax.dot_general(xb, w1_ref[...], dn,
                        preferred_element_type=jnp.float32)
    bq = lax.dot_general(xb, w3_ref[...], dn,
                         preferred_element_type=jnp.float32)
    h = a * jax.nn.sigmoid(a) * bq
    z_ref[...] = lax.dot_general(h, w2_ref[...], dn,
                                 preferred_element_type=jnp.float32)


def _shared(x, Ws1, Ws3, Ws2):
    return pl.pallas_call(
        _shared_body,
        grid=(T // SB,),
        in_specs=[
            pl.BlockSpec((SB, D), lambda i: (i, 0)),
            pl.BlockSpec((FS, D), lambda i: (0, 0)),
            pl.BlockSpec((FS, D), lambda i: (0, 0)),
            pl.BlockSpec((D, FS), lambda i: (0, 0)),
        ],
        out_specs=pl.BlockSpec((SB, D), lambda i: (i, 0)),
        out_shape=jax.ShapeDtypeStruct((T, D), jnp.float32),
    )(x, Ws1, Ws3, Ws2)


TPW = T // NW
TCH = 4
NCH5 = TPW // TCH
RCH = TCH * K
VL = 16


def _combine_body(ys_hbm, pos_hbm, w_hbm, z_hbm, y_hbm,
                  pos_v, w_v, rows_v, z_v, out_v, sem_g):
    wid = lax.axis_index("s") * NC + lax.axis_index("c")
    tbase = wid * TPW
    pbase = pl.multiple_of(tbase * K, 8)
    pltpu.sync_copy(pos_hbm.at[pl.ds(pbase, TPW * K)], pos_v)
    pltpu.sync_copy(w_hbm.at[pl.ds(pbase, TPW * K)], w_v)

    def chunk(ci, carry):
        t0 = tbase + ci * TCH
        p0 = pl.multiple_of(ci * RCH, RCH)
        pltpu.async_copy(ys_hbm.at[pos_v.at[pl.ds(p0, RCH)]], rows_v,
                         sem_g).wait()
        pltpu.sync_copy(z_hbm.at[pl.ds(t0, TCH)], z_v)
        for t in range(TCH):
            wb = [plsc.load_gather(
                      w_v, [jnp.full((VL,), ci * RCH + t * K + k, jnp.int32)])
                  for k in range(K)]

            def feat(v, c_):
                sl = pl.ds(pl.multiple_of(v * VL, VL), VL)
                acc = z_v[t, sl]
                for k in range(K):
                    acc = acc + wb[k] * rows_v[t * K + k, sl]
                out_v[t, sl] = acc
                return c_

            lax.fori_loop(0, D // VL, feat, 0)
        pltpu.sync_copy(out_v, y_hbm.at[pl.ds(t0, TCH)])
        return carry

    lax.fori_loop(0, NCH5, chunk, 0)


_combine = functools.partial(
    pl.kernel,
    out_type=jax.ShapeDtypeStruct((T, D), jnp.float32),
    mesh=plsc.VectorSubcoreMesh(core_axis_name="c", subcore_axis_name="s",
                                num_cores=NC, num_subcores=NS),
    scratch_types=[
        pltpu.VMEM((TPW * K,), jnp.int32),
        pltpu.VMEM((TPW * K,), jnp.float32),
        pltpu.VMEM((RCH, D), jnp.float32),
        pltpu.VMEM((TCH, D), jnp.float32),
        pltpu.VMEM((TCH, D), jnp.float32),
        pltpu.SemaphoreType.DMA,
    ],
)(_combine_body)


def kernel(x, gate_w, W1, W2, W3, Ws1, Ws2, Ws3):
    w8, idx8 = _gate(x, gate_w)
    pos_col, bexp, nb = _slots(idx8.reshape(P, 1))
    pos = pos_col.reshape(P)
    tok = jnp.repeat(jnp.arange(T, dtype=jnp.int32), K)
    xs = _dispatch(x, tok, pos)
    ys = _experts(xs, W1, W3, W2, bexp.reshape(NBLK), nb.reshape(1))
    z = _shared(x, Ws1, Ws3, Ws2)
    return _combine(ys, pos, w8.reshape(P), z)


# speedup vs baseline: 2.2835x; 2.2835x over previous
import functools

import jax
import jax.numpy as jnp
from jax import lax
from jax.experimental import pallas as pl
from jax.experimental.pallas import tpu as pltpu
from jax.experimental.pallas import tpu_sc as plsc

D = 1024
E = 64
K = 8
G = 8
KG = 4
F = 512
FS = 1024
T = 2048
P = T * K
BLK = 128
PAD = P + E * BLK
NBLK = PAD // BLK
TB = 256

NC, NS = 2, 16
NW = NC * NS


def _gate_body(x_ref, gwt_ref, w_ref, idx_ref):
    xb = x_ref[...]
    logits = jnp.dot(xb, gwt_ref[...], preferred_element_type=jnp.float32)
    m = jnp.max(logits, axis=-1, keepdims=True)
    ex = jnp.exp(logits - m)
    scores = ex / jnp.sum(ex, axis=-1, keepdims=True)

    lane64 = lax.broadcasted_iota(jnp.int32, (TB, E), 1)
    gid = lane64 // G
    neg = jnp.float32(-jnp.inf)

    gs_cols = []
    for g in range(G):
        gs_cols.append(jnp.max(jnp.where(gid == g, scores, neg), axis=-1,
                               keepdims=True))
    lane8 = lax.broadcasted_iota(jnp.int32, (TB, G), 1)
    gs = jnp.zeros((TB, G), jnp.float32)
    for g in range(G):
        gs = jnp.where(lane8 == g, gs_cols[g], gs)

    gmask = jnp.zeros((TB, G), jnp.float32)
    gm = gs
    big8 = jnp.int32(G + 1)
    for _ in range(KG):
        mx = jnp.max(gm, axis=-1, keepdims=True)
        sel = jnp.min(jnp.where(gm == mx, lane8, big8), axis=-1, keepdims=True)
        oh = lane8 == sel
        gmask = jnp.maximum(gmask, oh.astype(jnp.float32))
        gm = jnp.where(oh, neg, gm)

    emask = jnp.zeros((TB, E), jnp.float32)
    for g in range(G):
        emask = jnp.where(gid == g, gmask[:, g:g + 1], emask)

    masked = scores * emask
    big64 = jnp.int32(E + 1)
    wout = jnp.zeros((TB, K), jnp.float32)
    iout = jnp.zeros((TB, K), jnp.int32)
    lane_k = lax.broadcasted_iota(jnp.int32, (TB, K), 1)
    mm = masked
    for k in range(K):
        mx = jnp.max(mm, axis=-1, keepdims=True)
        sel = jnp.min(jnp.where(mm == mx, lane64, big64), axis=-1,
                      keepdims=True)
        oh = lane64 == sel
        wk = jnp.sum(jnp.where(oh, scores, 0.0), axis=-1, keepdims=True)
        mm = jnp.where(oh, neg, mm)
        wout = jnp.where(lane_k == k, wk, wout)
        iout = jnp.where(lane_k == k, sel, iout)
    w_ref[...] = wout
    idx_ref[...] = iout


def _gate(x, gate_w):
    gwt = gate_w.T
    return pl.pallas_call(
        _gate_body,
        grid=(T // TB,),
        in_specs=[
            pl.BlockSpec((TB, D), lambda i: (i, 0)),
            pl.BlockSpec((D, E), lambda i: (0, 0)),
        ],
        out_specs=[
            pl.BlockSpec((TB, K), lambda i: (i, 0)),
            pl.BlockSpec((TB, K), lambda i: (i, 0)),
        ],
        out_shape=[
            jax.ShapeDtypeStruct((T, K), jnp.float32),
            jax.ShapeDtypeStruct((T, K), jnp.int32),
        ],
    )(x, gwt)


CHUNK = 512
NCHUNK = P // CHUNK


def _slots_body(idx_ref, pos_ref, bexp_ref, nb_ref, part_ref):
    r = lax.broadcasted_iota(jnp.int32, (CHUNK, CHUNK), 0)
    c = lax.broadcasted_iota(jnp.int32, (CHUNK, CHUNK), 1)
    L = (r > c).astype(jnp.float32)
    lane64c = lax.broadcasted_iota(jnp.int32, (CHUNK, E), 1)

    def pass1(ci, run):
        sl = pl.ds(ci * CHUNK, CHUNK)
        oh = (idx_ref[sl, :] == lane64c).astype(jnp.float32)
        rank = jnp.dot(L, oh, preferred_element_type=jnp.float32)
        pick_rank = jnp.sum(rank * oh, axis=-1, keepdims=True)
        pick_prior = jnp.sum(run * oh, axis=-1, keepdims=True)
        part_ref[sl, :] = pick_rank + pick_prior
        return run + jnp.sum(oh, axis=0, keepdims=True)

    counts = lax.fori_loop(0, NCHUNK, pass1,
                           jnp.zeros((1, E), jnp.float32))

    counts_i = counts.astype(jnp.int32)
    padded_i = (counts_i + (BLK - 1)) // BLK * BLK
    padded = padded_i.astype(jnp.float32)
    ru = lax.broadcasted_iota(jnp.int32, (E, E), 0)
    cu = lax.broadcasted_iota(jnp.int32, (E, E), 1)
    U = (ru < cu).astype(jnp.float32)
    starts = jnp.dot(padded, U, preferred_element_type=jnp.float32)

    def pass2(ci, carry):
        sl = pl.ds(ci * CHUNK, CHUNK)
        oh = (idx_ref[sl, :] == lane64c).astype(jnp.float32)
        pick_start = jnp.sum(starts * oh, axis=-1, keepdims=True)
        pos_ref[sl, :] = (part_ref[sl, :] + pick_start).astype(jnp.int32)
        return carry

    lax.fori_loop(0, NCHUNK, pass2, 0)

    ends = starts + padded
    srow = (lax.broadcasted_iota(jnp.int32, (NBLK, E), 0) * BLK
            ).astype(jnp.float32)
    cnt = jnp.sum((ends <= srow).astype(jnp.float32), axis=-1,
                  keepdims=True)
    bexp_ref[...] = jnp.minimum(cnt.astype(jnp.int32), E - 1)
    nb_ref[...] = (jnp.sum(padded, axis=-1, keepdims=True)
                   / BLK).astype(jnp.int32)


def _slots(idx_col):
    return pl.pallas_call(
        _slots_body,
        out_shape=[
            jax.ShapeDtypeStruct((P, 1), jnp.int32),
            jax.ShapeDtypeStruct((NBLK, 1), jnp.int32),
            jax.ShapeDtypeStruct((1, 1), jnp.int32),
        ],
        scratch_shapes=[pltpu.VMEM((P, 1), jnp.float32)],
    )(idx_col)


CH3 = 32
PPW = P // NW
NCH3 = PPW // CH3


WREP = 128


def _dispatch_body(x_hbm, tok_hbm, slot_hbm, wrep_hbm, xs_hbm, ws_hbm,
                   tok_v, slot_v, rows_v, wrows_v, sem_g, sem_s, sem_w):
    wid = lax.axis_index("s") * NC + lax.axis_index("c")
    base = wid * PPW

    def chunk(i, carry):
        off = pl.multiple_of(base + i * CH3, CH3)
        pltpu.sync_copy(tok_hbm.at[pl.ds(off, CH3)], tok_v)
        pltpu.sync_copy(slot_hbm.at[pl.ds(off, CH3)], slot_v)
        pltpu.sync_copy(wrep_hbm.at[pl.ds(off, CH3)], wrows_v)
        pltpu.async_copy(x_hbm.at[tok_v], rows_v, sem_g).wait()
        pltpu.async_copy(rows_v, xs_hbm.at[slot_v], sem_s).wait()
        pltpu.async_copy(wrows_v, ws_hbm.at[slot_v], sem_w).wait()
        return carry

    lax.fori_loop(0, NCH3, chunk, 0)


@functools.cache
def _dispatch():
    return pl.kernel(
        _dispatch_body,
        out_type=(jax.ShapeDtypeStruct((PAD, D), jnp.float32),
                  jax.ShapeDtypeStruct((PAD, WREP), jnp.float32)),
        mesh=plsc.VectorSubcoreMesh(core_axis_name="c", subcore_axis_name="s",
                                    num_cores=NC, num_subcores=NS),
        scratch_types=[
            pltpu.VMEM((CH3,), jnp.int32),
            pltpu.VMEM((CH3,), jnp.int32),
            pltpu.VMEM((CH3, D), jnp.float32),
            pltpu.VMEM((CH3, WREP), jnp.float32),
            pltpu.SemaphoreType.DMA,
            pltpu.SemaphoreType.DMA,
            pltpu.SemaphoreType.DMA,
        ],
    )


def _expert_body(bexp_ref, nb_ref, xs_ref, ws_ref, w1_ref, w3_ref, w2_ref,
                 ys_ref):
    b = pl.program_id(0)

    @pl.when(b < nb_ref[0])
    def _():
        xb = xs_ref[...]
        w1 = w1_ref[0]
        w3 = w3_ref[0]
        w2 = w2_ref[0]
        dn = (((1,), (1,)), ((), ()))
        a = lax.dot_general(xb, w1, dn, preferred_element_type=jnp.float32)
        bq = lax.dot_general(xb, w3, dn, preferred_element_type=jnp.float32)
        h = a * jax.nn.sigmoid(a) * bq
        y = lax.dot_general(h, w2, dn, preferred_element_type=jnp.float32)
        ys_ref[...] = y * ws_ref[:, 0:1]


def _experts(xs, ws, W1, W3, W2, bexp, nb):
    def clamp(b, nb_):
        return jnp.minimum(b, nb_[0] - 1)

    grid_spec = pltpu.PrefetchScalarGridSpec(
        num_scalar_prefetch=2,
        grid=(NBLK,),
        in_specs=[
            pl.BlockSpec((BLK, D), lambda b, be, nb_: (clamp(b, nb_), 0)),
            pl.BlockSpec((BLK, WREP), lambda b, be, nb_: (clamp(b, nb_), 0)),
            pl.BlockSpec((1, F, D),
                         lambda b, be, nb_: (be[clamp(b, nb_)], 0, 0)),
            pl.BlockSpec((1, F, D),
                         lambda b, be, nb_: (be[clamp(b, nb_)], 0, 0)),
            pl.BlockSpec((1, D, F),
                         lambda b, be, nb_: (be[clamp(b, nb_)], 0, 0)),
        ],
        out_specs=pl.BlockSpec((BLK, D),
                               lambda b, be, nb_: (clamp(b, nb_), 0)),
    )
    return pl.pallas_call(
        _expert_body,
        grid_spec=grid_spec,
        out_shape=jax.ShapeDtypeStruct((PAD, D), jnp.float32),
    )(bexp, nb, xs, ws, W1, W3, W2)


SB = 128


def _shared_body(x_ref, w1_ref, w3_ref, w2_ref, z_ref):
    xb = x_ref[...]
    dn = (((1,), (1,)), ((), ()))
    a = lax.dot_general(xb, w1_ref[...], dn,
                        preferred_element_type=jnp.float32)
    bq = lax.dot_general(xb, w3_ref[...], dn,
                         preferred_element_type=jnp.float32)
    h = a * jax.nn.sigmoid(a) * bq
    z_ref[...] = lax.dot_general(h, w2_ref[...], dn,
                                 preferred_element_type=jnp.float32)


def _shared(x, Ws1, Ws3, Ws2):
    return pl.pallas_call(
        _shared_body,
        grid=(T // SB,),
        in_specs=[
            pl.BlockSpec((SB, D), lambda i: (i, 0)),
            pl.BlockSpec((FS, D), lambda i: (0, 0)),
            pl.BlockSpec((FS, D), lambda i: (0, 0)),
            pl.BlockSpec((D, FS), lambda i: (0, 0)),
        ],
        out_specs=pl.BlockSpec((SB, D), lambda i: (i, 0)),
        out_shape=jax.ShapeDtypeStruct((T, D), jnp.float32),
    )(x, Ws1, Ws3, Ws2)


TPW = T // NW
TCH = 4
NCH5 = TPW // TCH
RCH = TCH * K
VL = 16


def _combine_body(ys_hbm, pos_hbm, z_hbm, y_hbm,
                  pos_v, rows_v, z_v, out_v, sem_g):
    wid = lax.axis_index("s") * NC + lax.axis_index("c")
    tbase = wid * TPW
    pbase = pl.multiple_of(tbase * K, 8)
    pltpu.sync_copy(pos_hbm.at[pl.ds(pbase, TPW * K)], pos_v)

    def chunk(ci, carry):
        t0 = tbase + ci * TCH
        p0 = pl.multiple_of(ci * RCH, RCH)
        pltpu.async_copy(ys_hbm.at[pos_v.at[pl.ds(p0, RCH)]], rows_v,
                         sem_g).wait()
        pltpu.sync_copy(z_hbm.at[pl.ds(t0, TCH)], z_v)
        for t in range(TCH):

            def feat(v, c_):
                sl = pl.ds(pl.multiple_of(v * VL, VL), VL)
                acc = z_v[t, sl]
                for k in range(K):
                    acc = acc + rows_v[t * K + k, sl]
                out_v[t, sl] = acc
                return c_

            lax.fori_loop(0, D // VL, feat, 0)
        pltpu.sync_copy(out_v, y_hbm.at[pl.ds(t0, TCH)])
        return carry

    lax.fori_loop(0, NCH5, chunk, 0)


@functools.cache
def _combine():
    return pl.kernel(
        _combine_body,
        out_type=jax.ShapeDtypeStruct((T, D), jnp.float32),
        mesh=plsc.VectorSubcoreMesh(core_axis_name="c", subcore_axis_name="s",
                                    num_cores=NC, num_subcores=NS),
        scratch_types=[
            pltpu.VMEM((TPW * K,), jnp.int32),
            pltpu.VMEM((RCH, D), jnp.float32),
            pltpu.VMEM((TCH, D), jnp.float32),
            pltpu.VMEM((TCH, D), jnp.float32),
            pltpu.SemaphoreType.DMA,
        ],
    )


def kernel(x, gate_w, W1, W2, W3, Ws1, Ws2, Ws3):
    w8, idx8 = _gate(x, gate_w)
    pos_col, bexp, nb = _slots(idx8.reshape(P, 1))
    pos = pos_col.reshape(P)
    tok = jnp.repeat(jnp.arange(T, dtype=jnp.int32), K)
    wrep = jnp.broadcast_to(w8.reshape(P, 1), (P, WREP))
    xs, ws = _dispatch()(x, tok, pos, wrep)
    ys = _experts(xs, ws, W1, W3, W2, bexp.reshape(NBLK), nb.reshape(1))
    z = _shared(x, Ws1, Ws3, Ws2)
    return _combine()(ys, pos, z)
